# baseline (speedup 1.0000x reference)
import functools

import jax
import jax.numpy as jnp
from jax import lax
from jax.experimental import pallas as pl
from jax.experimental.pallas import tpu as pltpu
from jax.experimental.pallas import tpu_sc as plsc

NE = 3200000
DE = 16
DX = 128
N_NODES = 100000

RN = 1600
NRANGES = 64
NP = RN * NRANGES
WIN = 4000
NWIN = NE // WIN
NVREG = WIN // 16
FB = 1024
NSUB = FB // 128


def _sc_body(dst_hbm, fe_hbm, sums, mins, maxs, cnts,
             win, win2, idb, dlb, rows, asum, amin, amax, acnt, cnt_s, sem, semw):
    wid = lax.axis_index("s") * 2 + lax.axis_index("c")
    iota16 = lax.iota(jnp.int32, 16)
    ones16 = jnp.ones((16,), jnp.float32)
    zeros16 = jnp.zeros((16,), jnp.float32)
    pinf16 = jnp.full((16,), jnp.inf, jnp.float32)
    ninf16 = jnp.full((16,), -jnp.inf, jnp.float32)

    def init_idb(i, c):
        idb[pl.ds(pl.multiple_of(i * 16, 16), 16)] = iota16 + i * 16
        return c
    lax.fori_loop(0, FB // 16, init_idb, 0)

    trash16 = jnp.full((16,), RN, jnp.int32)

    def reset_dlb():
        def reset16(k, c):
            dlb[pl.ds(pl.multiple_of(k * 16, 16), 16)] = trash16
            return c
        lax.fori_loop(0, FB // 16, reset16, 0)

    def flush():
        copies = [
            pltpu.async_copy(
                fe_hbm.at[idb.at[pl.ds(j * 128, 128)]],
                rows.at[pl.ds(j * 128, 128)], sem)
            for j in range(NSUB)
        ]
        for c in copies:
            c.wait()

        def rmw16(k, c):
            dv = dlb[pl.ds(pl.multiple_of(k * 16, 16), 16)]
            for j in range(16):
                d = dv[j]
                e = k * 16 + j
                frow = rows[e]
                asum[d] = asum[d] + frow
                amin[d] = jnp.minimum(amin[d], frow)
                amax[d] = jnp.maximum(amax[d], frow)
                acnt[d] = acnt[d] + ones16
            return c
        lax.fori_loop(0, FB // 16, rmw16, 0)
        reset_dlb()
        cnt_s[0] = 0

    reset_dlb()

    for rpass in range(2):
        r = wid + rpass * 32
        lo = pl.multiple_of(r * RN, RN)

        def initb(i, c):
            asum[i] = zeros16
            amin[i] = pinf16
            amax[i] = ninf16
            acnt[i] = zeros16
            return c
        lax.fori_loop(0, RN + 1, initb, 0)
        cnt_s[0] = 0

        def scan_win(wbuf, w):
            def vreg_body(i, c2):
                v = wbuf[pl.ds(pl.multiple_of(i * 16, 16), 16)]
                m = (v >= lo) & (v < lo + RN)
                npop = plsc.all_reduce_population_count(m)

                @pl.when(npop[0] > 0)
                def _():
                    mi32 = m.astype(jnp.int32)
                    pos = plsc.cumsum(mi32) - mi32
                    cnt = cnt_s[0]
                    wpos = pos + cnt
                    idv = iota16 + (w * WIN + i * 16)
                    plsc.store_scatter(idb, [wpos], idv, mask=m)
                    plsc.store_scatter(dlb, [wpos], v - lo, mask=m)
                    cnt_s[0] = cnt + npop[0]

                    @pl.when(cnt_s[0] >= FB - 16)
                    def _():
                        flush()
                return c2
            lax.fori_loop(0, NVREG, vreg_body, 0)

        def wslice(w):
            return dst_hbm.at[pl.ds(pl.multiple_of(w * WIN, WIN), WIN)]

        pltpu.async_copy(wslice(0), win, semw)

        def win2_body(t, c):
            w0 = t * 2
            pltpu.make_async_copy(wslice(w0), win, semw).wait()
            pltpu.async_copy(wslice(w0 + 1), win2, semw)
            scan_win(win, w0)
            pltpu.make_async_copy(wslice(w0 + 1), win2, semw).wait()

            @pl.when(w0 + 2 < NWIN)
            def _():
                pltpu.async_copy(wslice(w0 + 2), win, semw)
            scan_win(win2, w0 + 1)
            return c
        lax.fori_loop(0, NWIN // 2, win2_body, 0)

        @pl.when(cnt_s[0] > 0)
        def _():
            flush()

        pltpu.sync_copy(asum.at[pl.ds(0, RN)], sums.at[pl.ds(lo, RN)])
        pltpu.sync_copy(amin.at[pl.ds(0, RN)], mins.at[pl.ds(lo, RN)])
        pltpu.sync_copy(amax.at[pl.ds(0, RN)], maxs.at[pl.ds(lo, RN)])
        pltpu.sync_copy(acnt.at[pl.ds(0, RN)], cnts.at[pl.ds(lo, RN)])


@jax.jit
def _scatter_sc(dst, fe):
    mesh = plsc.VectorSubcoreMesh(core_axis_name="c", subcore_axis_name="s")
    f = pl.kernel(
        _sc_body,
        out_type=[
            jax.ShapeDtypeStruct((NP, DE), jnp.float32),
            jax.ShapeDtypeStruct((NP, DE), jnp.float32),
            jax.ShapeDtypeStruct((NP, DE), jnp.float32),
            jax.ShapeDtypeStruct((NP, DE), jnp.float32),
        ],
        mesh=mesh,
        compiler_params=pltpu.CompilerParams(needs_layout_passes=False, use_tc_tiling_on_sc=False),
        scratch_types=[
            pltpu.VMEM((WIN,), jnp.int32),
            pltpu.VMEM((WIN,), jnp.int32),
            pltpu.VMEM((FB,), jnp.int32),
            pltpu.VMEM((FB,), jnp.int32),
            pltpu.VMEM((FB, DE), jnp.float32),
            pltpu.VMEM((RN + 1, DE), jnp.float32),
            pltpu.VMEM((RN + 1, DE), jnp.float32),
            pltpu.VMEM((RN + 1, DE), jnp.float32),
            pltpu.VMEM((RN + 1, DE), jnp.float32),
            pltpu.SMEM((1,), jnp.int32),
            pltpu.SemaphoreType.DMA,
            pltpu.SemaphoreType.DMA,
        ],
    )
    return f(dst, fe)


BM = 800


def _tc_body(sums_ref, mins_ref, maxs_ref, cnts_ref, wm_ref, wi_ref, wa_ref,
             b_ref, o_ref):
    cnt = cnts_ref[:, 0:1]
    has = cnt > 0.0
    denom = jnp.maximum(cnt, 1.0)
    me = jnp.where(has, sums_ref[...] / denom, 0.0)
    mi = jnp.where(has, mins_ref[...], 0.0)
    ma = jnp.where(has, maxs_ref[...], 0.0)
    acc = jnp.dot(me, wm_ref[...], preferred_element_type=jnp.float32)
    acc = acc + jnp.dot(mi, wi_ref[...], preferred_element_type=jnp.float32)
    acc = acc + jnp.dot(ma, wa_ref[...], preferred_element_type=jnp.float32)
    o_ref[...] = acc + b_ref[...]


@jax.jit
def _linear_tc(sums, mins, maxs, cnts, Wm, Wi, Wa, b2):
    part = lambda: pl.BlockSpec((BM, DE), lambda i: (i, 0))
    full = lambda shp: pl.BlockSpec(shp, lambda i: (0, 0))
    return pl.pallas_call(
        _tc_body,
        grid=(N_NODES // BM,),
        in_specs=[part(), part(), part(), part(),
                  full((DE, DX)), full((DE, DX)), full((DE, DX)),
                  full((1, DX))],
        out_specs=pl.BlockSpec((BM, DX), lambda i: (i, 0)),
        out_shape=jax.ShapeDtypeStruct((N_NODES, DX), jnp.float32),
    )(sums, mins, maxs, cnts, Wm, Wi, Wa, b2)


def kernel(fe, edge_index, W, b):
    dst = edge_index[1]
    sums, mins, maxs, cnts = _scatter_sc(dst, fe)
    Wm = jnp.transpose(W[:, :DE])
    Wi = jnp.transpose(W[:, DE:2 * DE])
    Wa = jnp.transpose(W[:, 2 * DE:])
    b2 = b.reshape(1, DX)
    return _linear_tc(sums, mins, maxs, cnts, Wm, Wi, Wa, b2)


# speedup vs baseline: 1.6567x; 1.6567x over previous
import functools

import jax
import jax.numpy as jnp
from jax import lax
from jax.experimental import pallas as pl
from jax.experimental.pallas import tpu as pltpu
from jax.experimental.pallas import tpu_sc as plsc

NE = 3200000
DE = 16
DX = 128
N_NODES = 100000

RN = 1600
NRANGES = 64
NP = RN * NRANGES
WIN = 3200
NWIN = NE // WIN
NVREG = WIN // 16
FB = 1024
NSUB = FB // 128


def _sc_body(dst_hbm, fe_hbm, sums, mins, maxs, cnts,
             win, win2, idb, dlb, rows, asum, amin, amax, acnt, cnt_s, sem, semw):
    wid = lax.axis_index("s") * 2 + lax.axis_index("c")
    iota16 = lax.iota(jnp.int32, 16)
    ones16 = jnp.ones((16,), jnp.float32)
    zeros16 = jnp.zeros((16,), jnp.float32)
    pinf16 = jnp.full((16,), jnp.inf, jnp.float32)
    ninf16 = jnp.full((16,), -jnp.inf, jnp.float32)

    def init_idb(i, c):
        idb[pl.ds(pl.multiple_of(i * 16, 16), 16)] = iota16 + i * 16
        return c
    lax.fori_loop(0, FB // 16, init_idb, 0)

    trash16 = jnp.full((16,), RN, jnp.int32)

    def reset_dlb():
        def reset16(k, c):
            dlb[pl.ds(pl.multiple_of(k * 16, 16), 16)] = trash16
            return c
        lax.fori_loop(0, FB // 16, reset16, 0)

    def flush():
        copies = [
            pltpu.async_copy(
                fe_hbm.at[idb.at[pl.ds(j * 128, 128)]],
                rows.at[pl.ds(j * 128, 128)], sem)
            for j in range(NSUB)
        ]
        for c in copies:
            c.wait()

        def rmw16(k, c):
            dv = dlb[pl.ds(pl.multiple_of(k * 16, 16), 16)]
            for j in range(16):
                d = dv[j]
                e = k * 16 + j
                frow = rows[e]
                asum[d] = asum[d] + frow
                amin[d] = jnp.minimum(amin[d], frow)
                amax[d] = jnp.maximum(amax[d], frow)
                acnt[d] = acnt[d] + ones16
            return c
        lax.fori_loop(0, FB // 16, rmw16, 0)
        reset_dlb()
        cnt_s[0] = 0

    reset_dlb()

    for rpass in range(2):
        r = wid + rpass * 32
        lo = pl.multiple_of(r * RN, RN)

        def initb(i, c):
            asum[i] = zeros16
            amin[i] = pinf16
            amax[i] = ninf16
            acnt[i] = zeros16
            return c
        lax.fori_loop(0, RN + 1, initb, 0)
        cnt_s[0] = 0

        def scan_win(wbuf, w):
            def blk_body(i, cntv):
                c = cntv
                for u in range(4):
                    off = pl.multiple_of((i * 4 + u) * 16, 16)
                    v = wbuf[pl.ds(off, 16)]
                    m = (v >= lo) & (v < lo + RN)
                    mi32 = m.astype(jnp.int32)
                    npop = plsc.all_reduce_population_count(m)
                    pos = plsc.cumsum(mi32) - mi32
                    wpos = pos + c
                    idv = iota16 + (w * WIN + (i * 4 + u) * 16)
                    plsc.store_scatter(idb, [wpos], idv, mask=m)
                    plsc.store_scatter(dlb, [wpos], v - lo, mask=m)
                    c = c + npop
                cnt_s[0] = c[0]

                @pl.when(cnt_s[0] >= FB - 64)
                def _():
                    flush()
                return jnp.full((16,), cnt_s[0], jnp.int32)
            lax.fori_loop(0, NVREG // 4, blk_body,
                          jnp.full((16,), cnt_s[0], jnp.int32))

        def wslice(w):
            return dst_hbm.at[pl.ds(pl.multiple_of(w * WIN, WIN), WIN)]

        pltpu.async_copy(wslice(0), win, semw)

        def win2_body(t, c):
            w0 = t * 2
            pltpu.make_async_copy(wslice(w0), win, semw).wait()
            pltpu.async_copy(wslice(w0 + 1), win2, semw)
            scan_win(win, w0)
            pltpu.make_async_copy(wslice(w0 + 1), win2, semw).wait()

            @pl.when(w0 + 2 < NWIN)
            def _():
                pltpu.async_copy(wslice(w0 + 2), win, semw)
            scan_win(win2, w0 + 1)
            return c
        lax.fori_loop(0, NWIN // 2, win2_body, 0)

        @pl.when(cnt_s[0] > 0)
        def _():
            flush()

        pltpu.sync_copy(asum.at[pl.ds(0, RN)], sums.at[pl.ds(lo, RN)])
        pltpu.sync_copy(amin.at[pl.ds(0, RN)], mins.at[pl.ds(lo, RN)])
        pltpu.sync_copy(amax.at[pl.ds(0, RN)], maxs.at[pl.ds(lo, RN)])
        pltpu.sync_copy(acnt.at[pl.ds(0, RN)], cnts.at[pl.ds(lo, RN)])


@jax.jit
def _scatter_sc(dst, fe):
    mesh = plsc.VectorSubcoreMesh(core_axis_name="c", subcore_axis_name="s")
    f = pl.kernel(
        _sc_body,
        out_type=[
            jax.ShapeDtypeStruct((NP, DE), jnp.float32),
            jax.ShapeDtypeStruct((NP, DE), jnp.float32),
            jax.ShapeDtypeStruct((NP, DE), jnp.float32),
            jax.ShapeDtypeStruct((NP, DE), jnp.float32),
        ],
        mesh=mesh,
        compiler_params=pltpu.CompilerParams(needs_layout_passes=False, use_tc_tiling_on_sc=False),
        scratch_types=[
            pltpu.VMEM((WIN,), jnp.int32),
            pltpu.VMEM((WIN,), jnp.int32),
            pltpu.VMEM((FB,), jnp.int32),
            pltpu.VMEM((FB,), jnp.int32),
            pltpu.VMEM((FB, DE), jnp.float32),
            pltpu.VMEM((RN + 1, DE), jnp.float32),
            pltpu.VMEM((RN + 1, DE), jnp.float32),
            pltpu.VMEM((RN + 1, DE), jnp.float32),
            pltpu.VMEM((RN + 1, DE), jnp.float32),
            pltpu.SMEM((1,), jnp.int32),
            pltpu.SemaphoreType.DMA,
            pltpu.SemaphoreType.DMA,
        ],
    )
    return f(dst, fe)


BM = 800


def _tc_body(sums_ref, mins_ref, maxs_ref, cnts_ref, wm_ref, wi_ref, wa_ref,
             b_ref, o_ref):
    cnt = cnts_ref[:, 0:1]
    has = cnt > 0.0
    denom = jnp.maximum(cnt, 1.0)
    me = jnp.where(has, sums_ref[...] / denom, 0.0)
    mi = jnp.where(has, mins_ref[...], 0.0)
    ma = jnp.where(has, maxs_ref[...], 0.0)
    acc = jnp.dot(me, wm_ref[...], preferred_element_type=jnp.float32)
    acc = acc + jnp.dot(mi, wi_ref[...], preferred_element_type=jnp.float32)
    acc = acc + jnp.dot(ma, wa_ref[...], preferred_element_type=jnp.float32)
    o_ref[...] = acc + b_ref[...]


@jax.jit
def _linear_tc(sums, mins, maxs, cnts, Wm, Wi, Wa, b2):
    part = lambda: pl.BlockSpec((BM, DE), lambda i: (i, 0))
    full = lambda shp: pl.BlockSpec(shp, lambda i: (0, 0))
    return pl.pallas_call(
        _tc_body,
        grid=(N_NODES // BM,),
        in_specs=[part(), part(), part(), part(),
                  full((DE, DX)), full((DE, DX)), full((DE, DX)),
                  full((1, DX))],
        out_specs=pl.BlockSpec((BM, DX), lambda i: (i, 0)),
        out_shape=jax.ShapeDtypeStruct((N_NODES, DX), jnp.float32),
    )(sums, mins, maxs, cnts, Wm, Wi, Wa, b2)


def kernel(fe, edge_index, W, b):
    dst = edge_index[1]
    sums, mins, maxs, cnts = _scatter_sc(dst, fe)
    Wm = jnp.transpose(W[:, :DE])
    Wi = jnp.transpose(W[:, DE:2 * DE])
    Wa = jnp.transpose(W[:, 2 * DE:])
    b2 = b.reshape(1, DX)
    return _linear_tc(sums, mins, maxs, cnts, Wm, Wi, Wa, b2)


# speedup vs baseline: 2.0528x; 1.2391x over previous
import functools

import jax
import jax.numpy as jnp
from jax import lax
from jax.experimental import pallas as pl
from jax.experimental.pallas import tpu as pltpu
from jax.experimental.pallas import tpu_sc as plsc

NE = 3200000
DE = 16
DX = 128
N_NODES = 100000

RN = 1600
NRANGES = 64
NP = RN * NRANGES
WIN = 3200
NWIN = NE // WIN
NVREG = WIN // 16
FB = 1024
FBL = FB // 16
NSUB = FB // 128


def _sc_body(dst_hbm, fe_hbm, sums, mins, maxs, cnts,
             win, win2, idb, dlb, rows, asum, amin, amax, acnt, cnt_s, sem, semw):
    wid = lax.axis_index("s") * 2 + lax.axis_index("c")
    iota16 = lax.iota(jnp.int32, 16)
    ones16 = jnp.ones((16,), jnp.float32)
    zeros16 = jnp.zeros((16,), jnp.float32)
    pinf16 = jnp.full((16,), jnp.inf, jnp.float32)
    ninf16 = jnp.full((16,), -jnp.inf, jnp.float32)
    zi16 = jnp.zeros((16,), jnp.int32)

    def init_idb(i, c):
        idb[pl.ds(pl.multiple_of(i * 16, 16), 16)] = iota16 + i * 16
        return c
    lax.fori_loop(0, FB // 16, init_idb, 0)

    trash16 = jnp.full((16,), RN, jnp.int32)

    def reset_dlb():
        def reset16(k, c):
            dlb[pl.ds(pl.multiple_of(k * 16, 16), 16)] = trash16
            return c
        lax.fori_loop(0, FB // 16, reset16, 0)

    def flush():
        copies = [
            pltpu.async_copy(
                fe_hbm.at[idb.at[pl.ds(j * 128, 128)]],
                rows.at[pl.ds(j * 128, 128)], sem)
            for j in range(NSUB)
        ]
        for c in copies:
            c.wait()

        def rmw16(k, c):
            dv = dlb[pl.ds(pl.multiple_of(k * 16, 16), 16)]
            for j in range(16):
                d = dv[j]
                e = k * 16 + j
                frow = rows[e]
                asum[d] = asum[d] + frow
                amin[d] = jnp.minimum(amin[d], frow)
                amax[d] = jnp.maximum(amax[d], frow)
                acnt[d] = acnt[d] + ones16
            return c
        lax.fori_loop(0, FB // 16, rmw16, 0)
        reset_dlb()

    reset_dlb()

    for rpass in range(2):
        r = wid + rpass * 32
        lo = pl.multiple_of(r * RN, RN)

        def initb(i, c):
            asum[i] = zeros16
            amin[i] = pinf16
            amax[i] = ninf16
            acnt[i] = zeros16
            return c
        lax.fori_loop(0, RN + 1, initb, 0)

        def scan_win(wbuf, w, cl):
            def blk_body(i, cl):
                for u in range(8):
                    off = pl.multiple_of((i * 8 + u) * 16, 16)
                    v = wbuf[pl.ds(off, 16)]
                    dl = v - lo
                    m = dl.astype(jnp.uint32) < jnp.uint32(RN)
                    slot = cl * 16 + iota16
                    idv = iota16 + (w * WIN + (i * 8 + u) * 16)
                    plsc.store_scatter(idb, [slot], idv, mask=m)
                    plsc.store_scatter(dlb, [slot], dl, mask=m)
                    cl = cl + m.astype(jnp.int32)
                mx = jnp.max(cl)

                @pl.when(mx >= FBL - 8)
                def _():
                    flush()
                return jnp.where(mx >= FBL - 8, zi16, cl)
            return lax.fori_loop(0, NVREG // 8, blk_body, cl)

        def wslice(w):
            return dst_hbm.at[pl.ds(pl.multiple_of(w * WIN, WIN), WIN)]

        pltpu.async_copy(wslice(0), win, semw)

        def win2_body(t, cl):
            w0 = t * 2
            pltpu.make_async_copy(wslice(w0), win, semw).wait()
            pltpu.async_copy(wslice(w0 + 1), win2, semw)
            cl = scan_win(win, w0, cl)
            pltpu.make_async_copy(wslice(w0 + 1), win2, semw).wait()

            @pl.when(w0 + 2 < NWIN)
            def _():
                pltpu.async_copy(wslice(w0 + 2), win, semw)
            cl = scan_win(win2, w0 + 1, cl)
            return cl
        lax.fori_loop(0, NWIN // 2, win2_body, zi16)

        flush()

        pltpu.sync_copy(asum.at[pl.ds(0, RN)], sums.at[pl.ds(lo, RN)])
        pltpu.sync_copy(amin.at[pl.ds(0, RN)], mins.at[pl.ds(lo, RN)])
        pltpu.sync_copy(amax.at[pl.ds(0, RN)], maxs.at[pl.ds(lo, RN)])
        pltpu.sync_copy(acnt.at[pl.ds(0, RN)], cnts.at[pl.ds(lo, RN)])


@jax.jit
def _scatter_sc(dst, fe):
    mesh = plsc.VectorSubcoreMesh(core_axis_name="c", subcore_axis_name="s")
    f = pl.kernel(
        _sc_body,
        out_type=[
            jax.ShapeDtypeStruct((NP, DE), jnp.float32),
            jax.ShapeDtypeStruct((NP, DE), jnp.float32),
            jax.ShapeDtypeStruct((NP, DE), jnp.float32),
            jax.ShapeDtypeStruct((NP, DE), jnp.float32),
        ],
        mesh=mesh,
        compiler_params=pltpu.CompilerParams(needs_layout_passes=False, use_tc_tiling_on_sc=False),
        scratch_types=[
            pltpu.VMEM((WIN,), jnp.int32),
            pltpu.VMEM((WIN,), jnp.int32),
            pltpu.VMEM((FB,), jnp.int32),
            pltpu.VMEM((FB,), jnp.int32),
            pltpu.VMEM((FB, DE), jnp.float32),
            pltpu.VMEM((RN + 1, DE), jnp.float32),
            pltpu.VMEM((RN + 1, DE), jnp.float32),
            pltpu.VMEM((RN + 1, DE), jnp.float32),
            pltpu.VMEM((RN + 1, DE), jnp.float32),
            pltpu.SMEM((1,), jnp.int32),
            pltpu.SemaphoreType.DMA,
            pltpu.SemaphoreType.DMA,
        ],
    )
    return f(dst, fe)


BM = 800


def _tc_body(sums_ref, mins_ref, maxs_ref, cnts_ref, wm_ref, wi_ref, wa_ref,
             b_ref, o_ref):
    cnt = cnts_ref[:, 0:1]
    has = cnt > 0.0
    denom = jnp.maximum(cnt, 1.0)
    me = jnp.where(has, sums_ref[...] / denom, 0.0)
    mi = jnp.where(has, mins_ref[...], 0.0)
    ma = jnp.where(has, maxs_ref[...], 0.0)
    acc = jnp.dot(me, wm_ref[...], preferred_element_type=jnp.float32)
    acc = acc + jnp.dot(mi, wi_ref[...], preferred_element_type=jnp.float32)
    acc = acc + jnp.dot(ma, wa_ref[...], preferred_element_type=jnp.float32)
    o_ref[...] = acc + b_ref[...]


@jax.jit
def _linear_tc(sums, mins, maxs, cnts, Wm, Wi, Wa, b2):
    part = lambda: pl.BlockSpec((BM, DE), lambda i: (i, 0))
    full = lambda shp: pl.BlockSpec(shp, lambda i: (0, 0))
    return pl.pallas_call(
        _tc_body,
        grid=(N_NODES // BM,),
        in_specs=[part(), part(), part(), part(),
                  full((DE, DX)), full((DE, DX)), full((DE, DX)),
                  full((1, DX))],
        out_specs=pl.BlockSpec((BM, DX), lambda i: (i, 0)),
        out_shape=jax.ShapeDtypeStruct((N_NODES, DX), jnp.float32),
    )(sums, mins, maxs, cnts, Wm, Wi, Wa, b2)


def kernel(fe, edge_index, W, b):
    dst = edge_index[1]
    sums, mins, maxs, cnts = _scatter_sc(dst, fe)
    Wm = jnp.transpose(W[:, :DE])
    Wi = jnp.transpose(W[:, DE:2 * DE])
    Wa = jnp.transpose(W[:, 2 * DE:])
    b2 = b.reshape(1, DX)
    return _linear_tc(sums, mins, maxs, cnts, Wm, Wi, Wa, b2)


# speedup vs baseline: 3.1260x; 1.5228x over previous
import functools

import jax
import jax.numpy as jnp
from jax import lax
from jax.experimental import pallas as pl
from jax.experimental.pallas import tpu as pltpu
from jax.experimental.pallas import tpu_sc as plsc

NE = 3200000
DE = 16
DX = 128
N_NODES = 100000

RN = 1600
NRANGES = 64
NP = RN * NRANGES
WIN = 3200
NWIN = NE // WIN
NVREG = WIN // 16
FB = 1024
FBL = FB // 16
NSUB = FB // 128


def _sc_body(dst_hbm, fe_hbm, sums, mins, maxs, cnts,
             win, win2, idb, dlb, rows, asum, amin, amax, acnt, cnt_s, sem, semw):
    wid = lax.axis_index("s") * 2 + lax.axis_index("c")
    iota16 = lax.iota(jnp.int32, 16)
    ones16 = jnp.ones((16,), jnp.float32)
    zeros16 = jnp.zeros((16,), jnp.float32)
    pinf16 = jnp.full((16,), jnp.inf, jnp.float32)
    ninf16 = jnp.full((16,), -jnp.inf, jnp.float32)
    zi16 = jnp.zeros((16,), jnp.int32)

    def init_idb(i, c):
        idb[pl.ds(pl.multiple_of(i * 16, 16), 16)] = iota16 + i * 16
        return c
    lax.fori_loop(0, FB // 16, init_idb, 0)

    trash16 = jnp.full((16,), RN, jnp.int32)

    def reset_dlb():
        def reset16(k, c):
            dlb[pl.ds(pl.multiple_of(k * 16, 16), 16)] = trash16
            return c
        lax.fori_loop(0, FB // 16, reset16, 0)

    def flush():
        copies = [
            pltpu.async_copy(
                fe_hbm.at[idb.at[pl.ds(j * 128, 128)]],
                rows.at[pl.ds(j * 128, 128)], sem)
            for j in range(NSUB)
        ]
        for c in copies:
            c.wait()

        def rmw16(k, c):
            dv = dlb[pl.ds(pl.multiple_of(k * 16, 16), 16)]
            for j in range(16):
                d = dv[j]
                e = k * 16 + j
                frow = rows[e]
                asum[d] = asum[d] + frow
                amin[d] = jnp.minimum(amin[d], frow)
                amax[d] = jnp.maximum(amax[d], frow)
                acnt[d] = acnt[d] + ones16
            return c
        lax.fori_loop(0, FB // 16, rmw16, 0)
        reset_dlb()

    reset_dlb()

    for rpass in range(2):
        r = wid + rpass * 32
        lo = pl.multiple_of(r * RN, RN)

        def initb(i, c):
            asum[i] = zeros16
            amin[i] = pinf16
            amax[i] = ninf16
            acnt[i] = zeros16
            return c
        lax.fori_loop(0, RN + 1, initb, 0)

        def scan_win(wbuf, w, cl):
            def blk_body(i, cl):
                vs = []
                for u in range(8):
                    off = pl.multiple_of((i * 8 + u) * 16, 16)
                    vs.append(wbuf[pl.ds(off, 16)])
                ms = [(v - lo).astype(jnp.uint32) < jnp.uint32(RN) for v in vs]
                for u in range(8):
                    v, m = vs[u], ms[u]
                    slot = cl * 16 + iota16
                    idv = iota16 + (w * WIN + (i * 8 + u) * 16)
                    plsc.store_scatter(idb, [slot], idv, mask=m)
                    plsc.store_scatter(dlb, [slot], v - lo, mask=m)
                    cl = cl + m.astype(jnp.int32)
                mx = jnp.max(cl)

                @pl.when(mx >= FBL - 8)
                def _():
                    flush()
                return jnp.where(mx >= FBL - 8, zi16, cl)
            return lax.fori_loop(0, NVREG // 8, blk_body, cl)

        def wslice(w):
            return dst_hbm.at[pl.ds(pl.multiple_of(w * WIN, WIN), WIN)]

        pltpu.async_copy(wslice(0), win, semw)

        def win2_body(t, cl):
            w0 = t * 2
            pltpu.make_async_copy(wslice(w0), win, semw).wait()
            pltpu.async_copy(wslice(w0 + 1), win2, semw)
            cl = scan_win(win, w0, cl)
            pltpu.make_async_copy(wslice(w0 + 1), win2, semw).wait()

            @pl.when(w0 + 2 < NWIN)
            def _():
                pltpu.async_copy(wslice(w0 + 2), win, semw)
            cl = scan_win(win2, w0 + 1, cl)
            return cl
        lax.fori_loop(0, NWIN // 2, win2_body, zi16)

        flush()

        pltpu.sync_copy(asum.at[pl.ds(0, RN)], sums.at[pl.ds(lo, RN)])
        pltpu.sync_copy(amin.at[pl.ds(0, RN)], mins.at[pl.ds(lo, RN)])
        pltpu.sync_copy(amax.at[pl.ds(0, RN)], maxs.at[pl.ds(lo, RN)])
        pltpu.sync_copy(acnt.at[pl.ds(0, RN)], cnts.at[pl.ds(lo, RN)])


@jax.jit
def _scatter_sc(dst, fe):
    mesh = plsc.VectorSubcoreMesh(core_axis_name="c", subcore_axis_name="s")
    f = pl.kernel(
        _sc_body,
        out_type=[
            jax.ShapeDtypeStruct((NP, DE), jnp.float32),
            jax.ShapeDtypeStruct((NP, DE), jnp.float32),
            jax.ShapeDtypeStruct((NP, DE), jnp.float32),
            jax.ShapeDtypeStruct((NP, DE), jnp.float32),
        ],
        mesh=mesh,
        compiler_params=pltpu.CompilerParams(needs_layout_passes=False, use_tc_tiling_on_sc=False),
        scratch_types=[
            pltpu.VMEM((WIN,), jnp.int32),
            pltpu.VMEM((WIN,), jnp.int32),
            pltpu.VMEM((FB,), jnp.int32),
            pltpu.VMEM((FB,), jnp.int32),
            pltpu.VMEM((FB, DE), jnp.float32),
            pltpu.VMEM((RN + 1, DE), jnp.float32),
            pltpu.VMEM((RN + 1, DE), jnp.float32),
            pltpu.VMEM((RN + 1, DE), jnp.float32),
            pltpu.VMEM((RN + 1, DE), jnp.float32),
            pltpu.SMEM((1,), jnp.int32),
            pltpu.SemaphoreType.DMA,
            pltpu.SemaphoreType.DMA,
        ],
    )
    return f(dst, fe)


BM = 800


def _tc_body(sums_ref, mins_ref, maxs_ref, cnts_ref, wm_ref, wi_ref, wa_ref,
             b_ref, o_ref):
    cnt = cnts_ref[:, 0:1]
    has = cnt > 0.0
    denom = jnp.maximum(cnt, 1.0)
    me = jnp.where(has, sums_ref[...] / denom, 0.0)
    mi = jnp.where(has, mins_ref[...], 0.0)
    ma = jnp.where(has, maxs_ref[...], 0.0)
    acc = jnp.dot(me, wm_ref[...], preferred_element_type=jnp.float32)
    acc = acc + jnp.dot(mi, wi_ref[...], preferred_element_type=jnp.float32)
    acc = acc + jnp.dot(ma, wa_ref[...], preferred_element_type=jnp.float32)
    o_ref[...] = acc + b_ref[...]


@jax.jit
def _linear_tc(sums, mins, maxs, cnts, Wm, Wi, Wa, b2):
    part = lambda: pl.BlockSpec((BM, DE), lambda i: (i, 0))
    full = lambda shp: pl.BlockSpec(shp, lambda i: (0, 0))
    return pl.pallas_call(
        _tc_body,
        grid=(N_NODES // BM,),
        in_specs=[part(), part(), part(), part(),
                  full((DE, DX)), full((DE, DX)), full((DE, DX)),
                  full((1, DX))],
        out_specs=pl.BlockSpec((BM, DX), lambda i: (i, 0)),
        out_shape=jax.ShapeDtypeStruct((N_NODES, DX), jnp.float32),
    )(sums, mins, maxs, cnts, Wm, Wi, Wa, b2)


def kernel(fe, edge_index, W, b):
    dst = edge_index[1]
    sums, mins, maxs, cnts = _scatter_sc(dst, fe)
    Wm = jnp.transpose(W[:, :DE])
    Wi = jnp.transpose(W[:, DE:2 * DE])
    Wa = jnp.transpose(W[:, 2 * DE:])
    b2 = b.reshape(1, DX)
    return _linear_tc(sums, mins, maxs, cnts, Wm, Wi, Wa, b2)
